# baseline (speedup 1.0000x reference)
import functools

import jax
import jax.numpy as jnp
from jax import lax
from jax.experimental import pallas as pl
from jax.experimental.pallas import tpu as pltpu
from jax.experimental.pallas import tpu_sc as plsc

_B = 1024
_NW = 32


def _cumsum_lanes(a, n):
    sh = 1
    while sh < n:
        z = jnp.zeros(a.shape[:1] + (sh,), a.dtype)
        a = a + jnp.concatenate([z, a[:, :-sh]], axis=1)
        sh *= 2
    return a


def _cumsum_subl(a, n):
    sh = 1
    while sh < n:
        z = jnp.zeros((sh,) + a.shape[1:], a.dtype)
        a = a + jnp.concatenate([z, a[:-sh, :]], axis=0)
        sh *= 2
    return a


def _router_body(x_ref, rw_ref, slots_ref, wnT_ref, be_ref, *, T, E, NB):
    x = x_ref[...]
    logits = jnp.dot(x, rw_ref[...].T, preferred_element_type=jnp.float32)
    pT = jax.nn.softmax(logits, axis=-1).T
    eiota = lax.broadcasted_iota(jnp.int32, (E, T), 0)
    m1 = jnp.max(pT, axis=0, keepdims=True)
    i1 = jnp.min(jnp.where(pT == m1, eiota, E), axis=0, keepdims=True)
    pm = jnp.where(eiota == i1, -jnp.inf, pT)
    m2 = jnp.max(pm, axis=0, keepdims=True)
    i2 = jnp.min(jnp.where(pm == m2, eiota, E), axis=0, keepdims=True)
    denom = m1 + m2
    wnT_ref[...] = jnp.concatenate([m1 / denom, m2 / denom], axis=0)

    sel = ((eiota == i1) | (eiota == i2)).astype(jnp.int32)
    csum = _cumsum_lanes(sel, T)
    cnt = csum[:, T - 1 : T]
    nblk = (cnt + (_B - 1)) // _B
    blk_incl = _cumsum_subl(nblk, E)
    base = _B * (blk_incl - nblk)
    v = base + csum - sel
    slot1 = jnp.sum(jnp.where(eiota == i1, v, 0), axis=0, keepdims=True)
    slot2 = jnp.sum(jnp.where(eiota == i2, v, 0), axis=0, keepdims=True)
    slots_ref[...] = jnp.concatenate([slot1, slot2], axis=0)

    biota = lax.broadcasted_iota(jnp.int32, (E, 128), 1)
    be = jnp.sum((biota >= blk_incl).astype(jnp.int32), axis=0, keepdims=True)
    be = jnp.minimum(be, E - 1)
    liota = lax.broadcasted_iota(jnp.int32, (1, 128), 1)
    be_ref[...] = jnp.where(liota == 127, blk_incl[E - 1 : E, :], be)


def _gmm_body(be_ref, xs_ref, w1_ref, w2_ref, bias_ref, ys_ref, *, H):
    b = pl.program_id(0)

    @pl.when(b < be_ref[127])
    def _live():
        h = jnp.dot(xs_ref[...], w1_ref[0].T, preferred_element_type=jnp.float32)
        g = h[:, :H]
        u = h[:, H:]
        act = g * jax.nn.sigmoid(g) * u
        y = jnp.dot(act, w2_ref[0].T, preferred_element_type=jnp.float32)
        ys_ref[...] = y + bias_ref[...][None, :]


def _combine_body(yk_ref, wnT_ref, out_ref):
    a = yk_ref[0]
    b = yk_ref[1]
    w1c = wnT_ref[0, :][:, None]
    w2c = wnT_ref[1, :][:, None]
    out_ref[...] = w1c * a + w2c * b


@jax.jit
def kernel(x, w1, w2, router_w, bias):
    T, D = x.shape
    E, H2, _ = w1.shape
    H = H2 // 2
    K = 2
    NB = (T * K + E * (_B - 1)) // _B
    S = NB * _B
    TPW = T // _NW
    HALF = TPW // 2

    slots, wnT, be = pl.pallas_call(
        functools.partial(_router_body, T=T, E=E, NB=NB),
        out_shape=(
            jax.ShapeDtypeStruct((K, T), jnp.int32),
            jax.ShapeDtypeStruct((K, T), jnp.float32),
            jax.ShapeDtypeStruct((1, 128), jnp.int32),
        ),
        in_specs=[
            pl.BlockSpec((T, D), lambda: (0, 0)),
            pl.BlockSpec((E, D), lambda: (0, 0)),
        ],
        out_specs=(
            pl.BlockSpec((K, T), lambda: (0, 0)),
            pl.BlockSpec((K, T), lambda: (0, 0)),
            pl.BlockSpec((1, 128), lambda: (0, 0)),
        ),
    )(x, router_w)

    mesh = plsc.VectorSubcoreMesh(core_axis_name="c", subcore_axis_name="s")

    @functools.partial(
        pl.kernel,
        mesh=mesh,
        out_type=jax.ShapeDtypeStruct((S, D), jnp.float32),
        scratch_types=[
            pltpu.VMEM((TPW,), jnp.int32),
            pltpu.VMEM((TPW,), jnp.int32),
            pltpu.VMEM((TPW, D), jnp.float32),
            pltpu.SemaphoreType.DMA,
            pltpu.SemaphoreType.DMA,
        ],
    )
    def _dispatch(xb_hbm, slots_hbm, xs_hbm, idx1_v, idx2_v, xbuf, sem1, sem2):
        wid = lax.axis_index("s") * 2 + lax.axis_index("c")
        base = wid * TPW
        pltpu.sync_copy(slots_hbm.at[0, pl.ds(base, TPW)], idx1_v)
        pltpu.sync_copy(slots_hbm.at[1, pl.ds(base, TPW)], idx2_v)
        pltpu.sync_copy(xb_hbm.at[pl.ds(base, TPW)], xbuf)
        c1 = pltpu.async_copy(xbuf, xs_hbm.at[idx1_v], sem1)
        c2 = pltpu.async_copy(xbuf, xs_hbm.at[idx2_v], sem2)
        c1.wait()
        c2.wait()

    xs = _dispatch(x, slots)

    grid_spec = pltpu.PrefetchScalarGridSpec(
        num_scalar_prefetch=1,
        grid=(NB,),
        in_specs=[
            pl.BlockSpec((_B, D), lambda b, be_s: (b, 0)),
            pl.BlockSpec((1, H2, D), lambda b, be_s: (be_s[b], 0, 0)),
            pl.BlockSpec((1, D, H), lambda b, be_s: (be_s[b], 0, 0)),
            pl.BlockSpec((D,), lambda b, be_s: (0,)),
        ],
        out_specs=pl.BlockSpec((_B, D), lambda b, be_s: (b, 0)),
    )
    ys = pl.pallas_call(
        functools.partial(_gmm_body, H=H),
        grid_spec=grid_spec,
        out_shape=jax.ShapeDtypeStruct((S, D), jnp.float32),
        compiler_params=pltpu.CompilerParams(
            dimension_semantics=("arbitrary",),
        ),
    )(be.reshape(128), xs, w1, w2, bias)

    @functools.partial(
        pl.kernel,
        mesh=mesh,
        out_type=jax.ShapeDtypeStruct((K, T, D), jnp.float32),
        scratch_types=[
            pltpu.VMEM((TPW,), jnp.int32),
            pltpu.VMEM((TPW,), jnp.int32),
            pltpu.VMEM((TPW, D), jnp.float32),
            pltpu.VMEM((TPW, D), jnp.float32),
            pltpu.SemaphoreType.DMA,
            pltpu.SemaphoreType.DMA,
        ],
    )
    def _unperm(ys_hbm, slots_hbm, yk_hbm, idx1_v, idx2_v, buf1, buf2, sem1, sem2):
        wid = lax.axis_index("s") * 2 + lax.axis_index("c")
        base = wid * TPW
        pltpu.sync_copy(slots_hbm.at[0, pl.ds(base, TPW)], idx1_v)
        pltpu.sync_copy(slots_hbm.at[1, pl.ds(base, TPW)], idx2_v)
        c1 = pltpu.async_copy(ys_hbm.at[idx1_v], buf1, sem1)
        c2 = pltpu.async_copy(ys_hbm.at[idx2_v], buf2, sem2)
        c1.wait()
        pltpu.sync_copy(buf1, yk_hbm.at[0, pl.ds(base, TPW)])
        c2.wait()
        pltpu.sync_copy(buf2, yk_hbm.at[1, pl.ds(base, TPW)])

    yk = _unperm(ys, slots)

    BT = 256
    out = pl.pallas_call(
        _combine_body,
        grid=(T // BT,),
        out_shape=jax.ShapeDtypeStruct((T, D), jnp.float32),
        in_specs=[
            pl.BlockSpec((K, BT, D), lambda t: (0, t, 0)),
            pl.BlockSpec((K, BT), lambda t: (0, t)),
        ],
        out_specs=pl.BlockSpec((BT, D), lambda t: (t, 0)),
    )(yk, wnT)
    return out


# speedup vs baseline: 1.1050x; 1.1050x over previous
import functools

import jax
import jax.numpy as jnp
from jax import lax
from jax.experimental import pallas as pl
from jax.experimental.pallas import tpu as pltpu
from jax.experimental.pallas import tpu_sc as plsc

_B = 576
_NW = 32


def _cumsum_lanes(a, n):
    sh = 1
    while sh < n:
        z = jnp.zeros(a.shape[:1] + (sh,), a.dtype)
        a = a + jnp.concatenate([z, a[:, :-sh]], axis=1)
        sh *= 2
    return a


def _cumsum_subl(a, n):
    sh = 1
    while sh < n:
        z = jnp.zeros((sh,) + a.shape[1:], a.dtype)
        a = a + jnp.concatenate([z, a[:-sh, :]], axis=0)
        sh *= 2
    return a


def _router_body(x_ref, rw_ref, slots_ref, wnT_ref, be_ref, *, T, E, NB):
    x = x_ref[...]
    logits = jnp.dot(x, rw_ref[...].T, preferred_element_type=jnp.float32)
    pT = jax.nn.softmax(logits, axis=-1).T
    eiota = lax.broadcasted_iota(jnp.int32, (E, T), 0)
    m1 = jnp.max(pT, axis=0, keepdims=True)
    i1 = jnp.min(jnp.where(pT == m1, eiota, E), axis=0, keepdims=True)
    pm = jnp.where(eiota == i1, -jnp.inf, pT)
    m2 = jnp.max(pm, axis=0, keepdims=True)
    i2 = jnp.min(jnp.where(pm == m2, eiota, E), axis=0, keepdims=True)
    denom = m1 + m2
    wnT_ref[...] = jnp.concatenate([m1 / denom, m2 / denom], axis=0)

    sel = ((eiota == i1) | (eiota == i2)).astype(jnp.int32)
    csum = _cumsum_lanes(sel, T)
    cnt = csum[:, T - 1 : T]
    nblk = (cnt + (_B - 1)) // _B
    blk_incl = _cumsum_subl(nblk, E)
    base = _B * (blk_incl - nblk)
    v = base + csum - sel
    slot1 = jnp.sum(jnp.where(eiota == i1, v, 0), axis=0, keepdims=True)
    slot2 = jnp.sum(jnp.where(eiota == i2, v, 0), axis=0, keepdims=True)
    slots_ref[...] = jnp.concatenate([slot1, slot2], axis=0)

    biota = lax.broadcasted_iota(jnp.int32, (E, 128), 1)
    be = jnp.sum((biota >= blk_incl).astype(jnp.int32), axis=0, keepdims=True)
    be = jnp.minimum(be, E - 1)
    liota = lax.broadcasted_iota(jnp.int32, (1, 128), 1)
    be_ref[...] = jnp.where(liota == 127, blk_incl[E - 1 : E, :], be)


def _gmm_body(be_ref, xs_ref, w1_ref, w2_ref, bias_ref, ys_ref, *, H):
    b = pl.program_id(0)

    @pl.when(b < be_ref[127])
    def _live():
        h = jnp.dot(xs_ref[...], w1_ref[0].T, preferred_element_type=jnp.float32)
        g = h[:, :H]
        u = h[:, H:]
        act = g * jax.nn.sigmoid(g) * u
        y = jnp.dot(act, w2_ref[0].T, preferred_element_type=jnp.float32)
        ys_ref[...] = y + bias_ref[...][None, :]


def _combine_body(yk_ref, wnT_ref, out_ref):
    a = yk_ref[0]
    b = yk_ref[1]
    w1c = wnT_ref[0, :][:, None]
    w2c = wnT_ref[1, :][:, None]
    out_ref[...] = w1c * a + w2c * b


@jax.jit
def kernel(x, w1, w2, router_w, bias):
    T, D = x.shape
    E, H2, _ = w1.shape
    H = H2 // 2
    K = 2
    NB = (T * K + E * (_B - 1)) // _B
    S = NB * _B
    TPW = T // _NW
    HALF = TPW // 2

    slots, wnT, be = pl.pallas_call(
        functools.partial(_router_body, T=T, E=E, NB=NB),
        out_shape=(
            jax.ShapeDtypeStruct((K, T), jnp.int32),
            jax.ShapeDtypeStruct((K, T), jnp.float32),
            jax.ShapeDtypeStruct((1, 128), jnp.int32),
        ),
        in_specs=[
            pl.BlockSpec((T, D), lambda: (0, 0)),
            pl.BlockSpec((E, D), lambda: (0, 0)),
        ],
        out_specs=(
            pl.BlockSpec((K, T), lambda: (0, 0)),
            pl.BlockSpec((K, T), lambda: (0, 0)),
            pl.BlockSpec((1, 128), lambda: (0, 0)),
        ),
    )(x, router_w)

    mesh = plsc.VectorSubcoreMesh(core_axis_name="c", subcore_axis_name="s")

    @functools.partial(
        pl.kernel,
        mesh=mesh,
        out_type=jax.ShapeDtypeStruct((S, D), jnp.float32),
        scratch_types=[
            pltpu.VMEM((TPW,), jnp.int32),
            pltpu.VMEM((TPW,), jnp.int32),
            pltpu.VMEM((TPW, D), jnp.float32),
            pltpu.SemaphoreType.DMA,
            pltpu.SemaphoreType.DMA,
        ],
    )
    def _dispatch(xb_hbm, slots_hbm, xs_hbm, idx1_v, idx2_v, xbuf, sem1, sem2):
        wid = lax.axis_index("s") * 2 + lax.axis_index("c")
        base = wid * TPW
        pltpu.sync_copy(slots_hbm.at[0, pl.ds(base, TPW)], idx1_v)
        pltpu.sync_copy(slots_hbm.at[1, pl.ds(base, TPW)], idx2_v)
        pltpu.sync_copy(xb_hbm.at[pl.ds(base, TPW)], xbuf)
        c1 = pltpu.async_copy(xbuf, xs_hbm.at[idx1_v], sem1)
        c2 = pltpu.async_copy(xbuf, xs_hbm.at[idx2_v], sem2)
        c1.wait()
        c2.wait()

    xs = _dispatch(x, slots)

    grid_spec = pltpu.PrefetchScalarGridSpec(
        num_scalar_prefetch=1,
        grid=(NB,),
        in_specs=[
            pl.BlockSpec((_B, D), lambda b, be_s: (b, 0)),
            pl.BlockSpec((1, H2, D), lambda b, be_s: (be_s[b], 0, 0)),
            pl.BlockSpec((1, D, H), lambda b, be_s: (be_s[b], 0, 0)),
            pl.BlockSpec((D,), lambda b, be_s: (0,)),
        ],
        out_specs=pl.BlockSpec((_B, D), lambda b, be_s: (b, 0)),
    )
    ys = pl.pallas_call(
        functools.partial(_gmm_body, H=H),
        grid_spec=grid_spec,
        out_shape=jax.ShapeDtypeStruct((S, D), jnp.float32),
        compiler_params=pltpu.CompilerParams(
            dimension_semantics=("arbitrary",),
        ),
    )(be.reshape(128), xs, w1, w2, bias)

    @functools.partial(
        pl.kernel,
        mesh=mesh,
        out_type=jax.ShapeDtypeStruct((K, T, D), jnp.float32),
        scratch_types=[
            pltpu.VMEM((TPW,), jnp.int32),
            pltpu.VMEM((TPW,), jnp.int32),
            pltpu.VMEM((TPW, D), jnp.float32),
            pltpu.VMEM((TPW, D), jnp.float32),
            pltpu.SemaphoreType.DMA,
            pltpu.SemaphoreType.DMA,
        ],
    )
    def _unperm(ys_hbm, slots_hbm, yk_hbm, idx1_v, idx2_v, buf1, buf2, sem1, sem2):
        wid = lax.axis_index("s") * 2 + lax.axis_index("c")
        base = wid * TPW
        pltpu.sync_copy(slots_hbm.at[0, pl.ds(base, TPW)], idx1_v)
        pltpu.sync_copy(slots_hbm.at[1, pl.ds(base, TPW)], idx2_v)
        c1 = pltpu.async_copy(ys_hbm.at[idx1_v], buf1, sem1)
        c2 = pltpu.async_copy(ys_hbm.at[idx2_v], buf2, sem2)
        c1.wait()
        pltpu.sync_copy(buf1, yk_hbm.at[0, pl.ds(base, TPW)])
        c2.wait()
        pltpu.sync_copy(buf2, yk_hbm.at[1, pl.ds(base, TPW)])

    yk = _unperm(ys, slots)

    BT = 256
    out = pl.pallas_call(
        _combine_body,
        grid=(T // BT,),
        out_shape=jax.ShapeDtypeStruct((T, D), jnp.float32),
        in_specs=[
            pl.BlockSpec((K, BT, D), lambda t: (0, t, 0)),
            pl.BlockSpec((K, BT), lambda t: (0, t)),
        ],
        out_specs=pl.BlockSpec((BT, D), lambda t: (t, 0)),
    )(yk, wnT)
    return out


# speedup vs baseline: 1.1946x; 1.0811x over previous
import functools

import jax
import jax.numpy as jnp
from jax import lax
from jax.experimental import pallas as pl
from jax.experimental.pallas import tpu as pltpu
from jax.experimental.pallas import tpu_sc as plsc

_B = 576
_NW = 32


def _cumsum_lanes(a, n):
    sh = 1
    while sh < n:
        z = jnp.zeros(a.shape[:1] + (sh,), a.dtype)
        a = a + jnp.concatenate([z, a[:, :-sh]], axis=1)
        sh *= 2
    return a


def _cumsum_subl(a, n):
    sh = 1
    while sh < n:
        z = jnp.zeros((sh,) + a.shape[1:], a.dtype)
        a = a + jnp.concatenate([z, a[:-sh, :]], axis=0)
        sh *= 2
    return a


def _router_body(x_ref, rw_ref, slots_ref, wnT_ref, be_ref, *, T, E, NB):
    x = x_ref[...]
    logits = jnp.dot(x, rw_ref[...].T, preferred_element_type=jnp.float32)
    pT = jax.nn.softmax(logits, axis=-1).T
    eiota = lax.broadcasted_iota(jnp.int32, (E, T), 0)
    m1 = jnp.max(pT, axis=0, keepdims=True)
    i1 = jnp.min(jnp.where(pT == m1, eiota, E), axis=0, keepdims=True)
    pm = jnp.where(eiota == i1, -jnp.inf, pT)
    m2 = jnp.max(pm, axis=0, keepdims=True)
    i2 = jnp.min(jnp.where(pm == m2, eiota, E), axis=0, keepdims=True)
    denom = m1 + m2
    wnT_ref[...] = jnp.concatenate([m1 / denom, m2 / denom], axis=0)

    sel = ((eiota == i1) | (eiota == i2)).astype(jnp.int32)
    csum = _cumsum_lanes(sel, T)
    cnt = csum[:, T - 1 : T]
    nblk = (cnt + (_B - 1)) // _B
    blk_incl = _cumsum_subl(nblk, E)
    base = _B * (blk_incl - nblk)
    v = base + csum - sel
    slot1 = jnp.sum(jnp.where(eiota == i1, v, 0), axis=0, keepdims=True)
    slot2 = jnp.sum(jnp.where(eiota == i2, v, 0), axis=0, keepdims=True)
    slots_ref[...] = jnp.concatenate([slot1, slot2], axis=0)

    biota = lax.broadcasted_iota(jnp.int32, (E, 128), 1)
    be = jnp.sum((biota >= blk_incl).astype(jnp.int32), axis=0, keepdims=True)
    be = jnp.minimum(be, E - 1)
    liota = lax.broadcasted_iota(jnp.int32, (1, 128), 1)
    be_ref[...] = jnp.where(liota == 127, blk_incl[E - 1 : E, :], be)


def _gmm_body(be_ref, xs_ref, w1_ref, w2_ref, bias_ref, ys_ref, *, H):
    b = pl.program_id(0)

    @pl.when(b < be_ref[127])
    def _live():
        h = jnp.dot(xs_ref[...], w1_ref[0].T, preferred_element_type=jnp.float32)
        g = h[:, :H]
        u = h[:, H:]
        act = g * jax.nn.sigmoid(g) * u
        y = jnp.dot(act, w2_ref[0].T, preferred_element_type=jnp.float32)
        ys_ref[...] = y + bias_ref[...][None, :]


def _combine_body(yk_ref, wnT_ref, out_ref):
    a = yk_ref[0]
    b = yk_ref[1]
    w1c = wnT_ref[0, :][:, None]
    w2c = wnT_ref[1, :][:, None]
    out_ref[...] = w1c * a + w2c * b


@jax.jit
def kernel(x, w1, w2, router_w, bias):
    T, D = x.shape
    E, H2, _ = w1.shape
    H = H2 // 2
    K = 2
    NB = (T * K + E * (_B - 1)) // _B
    S = NB * _B
    TPW = T // _NW
    HALF = TPW // 2

    slots, wnT, be = pl.pallas_call(
        functools.partial(_router_body, T=T, E=E, NB=NB),
        out_shape=(
            jax.ShapeDtypeStruct((K, T), jnp.int32),
            jax.ShapeDtypeStruct((K, T), jnp.float32),
            jax.ShapeDtypeStruct((1, 128), jnp.int32),
        ),
        in_specs=[
            pl.BlockSpec((T, D), lambda: (0, 0)),
            pl.BlockSpec((E, D), lambda: (0, 0)),
        ],
        out_specs=(
            pl.BlockSpec((K, T), lambda: (0, 0)),
            pl.BlockSpec((K, T), lambda: (0, 0)),
            pl.BlockSpec((1, 128), lambda: (0, 0)),
        ),
    )(x, router_w)

    mesh = plsc.VectorSubcoreMesh(core_axis_name="c", subcore_axis_name="s")

    @functools.partial(
        pl.kernel,
        mesh=mesh,
        out_type=jax.ShapeDtypeStruct((S, D), jnp.float32),
        scratch_types=[
            pltpu.VMEM((TPW,), jnp.int32),
            pltpu.VMEM((TPW,), jnp.int32),
            pltpu.VMEM((TPW, D), jnp.float32),
            pltpu.SemaphoreType.DMA,
            pltpu.SemaphoreType.DMA,
        ],
    )
    def _dispatch(xb_hbm, slots_hbm, xs_hbm, idx1_v, idx2_v, xbuf, sem1, sem2):
        wid = lax.axis_index("s") * 2 + lax.axis_index("c")
        base = wid * TPW
        pltpu.sync_copy(slots_hbm.at[0, pl.ds(base, TPW)], idx1_v)
        pltpu.sync_copy(slots_hbm.at[1, pl.ds(base, TPW)], idx2_v)
        pltpu.sync_copy(xb_hbm.at[pl.ds(base, TPW)], xbuf)
        c1 = pltpu.async_copy(xbuf, xs_hbm.at[idx1_v], sem1)
        c2 = pltpu.async_copy(xbuf, xs_hbm.at[idx2_v], sem2)
        c1.wait()
        c2.wait()

    xs = _dispatch(x, slots)

    grid_spec = pltpu.PrefetchScalarGridSpec(
        num_scalar_prefetch=1,
        grid=(NB,),
        in_specs=[
            pl.BlockSpec(
                (_B, D),
                lambda b, be_s: (jnp.minimum(b, be_s[127] - 1), 0)),
            pl.BlockSpec(
                (1, H2, D),
                lambda b, be_s: (be_s[jnp.minimum(b, be_s[127] - 1)], 0, 0)),
            pl.BlockSpec(
                (1, D, H),
                lambda b, be_s: (be_s[jnp.minimum(b, be_s[127] - 1)], 0, 0)),
            pl.BlockSpec((D,), lambda b, be_s: (0,)),
        ],
        out_specs=pl.BlockSpec(
            (_B, D),
            lambda b, be_s: (jnp.where(b < be_s[127], b, NB - 1), 0)),
    )
    ys = pl.pallas_call(
        functools.partial(_gmm_body, H=H),
        grid_spec=grid_spec,
        out_shape=jax.ShapeDtypeStruct((S, D), jnp.float32),
        compiler_params=pltpu.CompilerParams(
            dimension_semantics=("arbitrary",),
        ),
    )(be.reshape(128), xs, w1, w2, bias)

    @functools.partial(
        pl.kernel,
        mesh=mesh,
        out_type=jax.ShapeDtypeStruct((K, T, D), jnp.float32),
        scratch_types=[
            pltpu.VMEM((TPW,), jnp.int32),
            pltpu.VMEM((TPW,), jnp.int32),
            pltpu.VMEM((TPW, D), jnp.float32),
            pltpu.VMEM((TPW, D), jnp.float32),
            pltpu.SemaphoreType.DMA,
            pltpu.SemaphoreType.DMA,
        ],
    )
    def _unperm(ys_hbm, slots_hbm, yk_hbm, idx1_v, idx2_v, buf1, buf2, sem1, sem2):
        wid = lax.axis_index("s") * 2 + lax.axis_index("c")
        base = wid * TPW
        pltpu.sync_copy(slots_hbm.at[0, pl.ds(base, TPW)], idx1_v)
        pltpu.sync_copy(slots_hbm.at[1, pl.ds(base, TPW)], idx2_v)
        c1 = pltpu.async_copy(ys_hbm.at[idx1_v], buf1, sem1)
        c2 = pltpu.async_copy(ys_hbm.at[idx2_v], buf2, sem2)
        c1.wait()
        pltpu.sync_copy(buf1, yk_hbm.at[0, pl.ds(base, TPW)])
        c2.wait()
        pltpu.sync_copy(buf2, yk_hbm.at[1, pl.ds(base, TPW)])

    yk = _unperm(ys, slots)

    BT = 256
    out = pl.pallas_call(
        _combine_body,
        grid=(T // BT,),
        out_shape=jax.ShapeDtypeStruct((T, D), jnp.float32),
        in_specs=[
            pl.BlockSpec((K, BT, D), lambda t: (0, t, 0)),
            pl.BlockSpec((K, BT), lambda t: (0, t)),
        ],
        out_specs=pl.BlockSpec((BT, D), lambda t: (t, 0)),
    )(yk, wnT)
    return out


# speedup vs baseline: 1.2454x; 1.0425x over previous
import functools

import jax
import jax.numpy as jnp
from jax import lax
from jax.experimental import pallas as pl
from jax.experimental.pallas import tpu as pltpu
from jax.experimental.pallas import tpu_sc as plsc

_B = 576
_NW = 32


def _cumsum_lanes(a, n):
    sh = 1
    while sh < n:
        z = jnp.zeros(a.shape[:1] + (sh,), a.dtype)
        a = a + jnp.concatenate([z, a[:, :-sh]], axis=1)
        sh *= 2
    return a


def _cumsum_subl(a, n):
    sh = 1
    while sh < n:
        z = jnp.zeros((sh,) + a.shape[1:], a.dtype)
        a = a + jnp.concatenate([z, a[:-sh, :]], axis=0)
        sh *= 2
    return a


def _router_body(x_ref, rw_ref, slots_ref, wnT_ref, be_ref, xpk_ref, *, T, E, NB):
    x = x_ref[...]
    Dh = x.shape[1] // 2
    u = lax.bitcast_convert_type(x, jnp.uint32)
    r = (u + jnp.uint32(0x7FFF) + ((u >> 16) & jnp.uint32(1))) >> 16
    xpk_ref[...] = lax.bitcast_convert_type(
        r[:, :Dh] | (r[:, Dh:] << 16), jnp.int32)
    logits = jnp.dot(x, rw_ref[...].T, preferred_element_type=jnp.float32)
    pT = jax.nn.softmax(logits, axis=-1).T
    eiota = lax.broadcasted_iota(jnp.int32, (E, T), 0)
    m1 = jnp.max(pT, axis=0, keepdims=True)
    i1 = jnp.min(jnp.where(pT == m1, eiota, E), axis=0, keepdims=True)
    pm = jnp.where(eiota == i1, -jnp.inf, pT)
    m2 = jnp.max(pm, axis=0, keepdims=True)
    i2 = jnp.min(jnp.where(pm == m2, eiota, E), axis=0, keepdims=True)
    denom = m1 + m2
    wnT_ref[...] = jnp.concatenate([m1 / denom, m2 / denom], axis=0)

    sel = ((eiota == i1) | (eiota == i2)).astype(jnp.int32)
    csum = _cumsum_lanes(sel, T)
    cnt = csum[:, T - 1 : T]
    nblk = (cnt + (_B - 1)) // _B
    blk_incl = _cumsum_subl(nblk, E)
    base = _B * (blk_incl - nblk)
    v = base + csum - sel
    slot1 = jnp.sum(jnp.where(eiota == i1, v, 0), axis=0, keepdims=True)
    slot2 = jnp.sum(jnp.where(eiota == i2, v, 0), axis=0, keepdims=True)
    slots_ref[...] = jnp.concatenate([slot1, slot2], axis=0)

    biota = lax.broadcasted_iota(jnp.int32, (E, 128), 1)
    be = jnp.sum((biota >= blk_incl).astype(jnp.int32), axis=0, keepdims=True)
    be = jnp.minimum(be, E - 1)
    liota = lax.broadcasted_iota(jnp.int32, (1, 128), 1)
    be_ref[...] = jnp.where(liota == 127, blk_incl[E - 1 : E, :], be)


def _gmm_body(be_ref, xs_ref, w1_ref, w2_ref, bias_ref, ys_ref, *, H):
    b = pl.program_id(0)

    @pl.when(b < be_ref[127])
    def _live():
        p = lax.bitcast_convert_type(xs_ref[...], jnp.uint32)
        left = lax.bitcast_convert_type(p << 16, jnp.float32)
        right = lax.bitcast_convert_type(
            p & jnp.uint32(0xFFFF0000), jnp.float32)
        xb = jnp.concatenate([left, right], axis=1)
        h = jnp.dot(xb, w1_ref[0].T, preferred_element_type=jnp.float32)
        g = h[:, :H]
        u = h[:, H:]
        act = g * jax.nn.sigmoid(g) * u
        y = jnp.dot(act, w2_ref[0].T, preferred_element_type=jnp.float32)
        ys_ref[...] = y + bias_ref[...][None, :]


def _combine_body(yk_ref, wnT_ref, out_ref):
    a = yk_ref[0]
    b = yk_ref[1]
    w1c = wnT_ref[0, :][:, None]
    w2c = wnT_ref[1, :][:, None]
    out_ref[...] = w1c * a + w2c * b


@jax.jit
def kernel(x, w1, w2, router_w, bias):
    T, D = x.shape
    E, H2, _ = w1.shape
    H = H2 // 2
    K = 2
    NB = (T * K + E * (_B - 1)) // _B
    S = NB * _B
    TPW = T // _NW
    HALF = TPW // 2

    D2 = D // 2
    slots, wnT, be, xpk = pl.pallas_call(
        functools.partial(_router_body, T=T, E=E, NB=NB),
        out_shape=(
            jax.ShapeDtypeStruct((K, T), jnp.int32),
            jax.ShapeDtypeStruct((K, T), jnp.float32),
            jax.ShapeDtypeStruct((1, 128), jnp.int32),
            jax.ShapeDtypeStruct((T, D2), jnp.int32),
        ),
        in_specs=[
            pl.BlockSpec((T, D), lambda: (0, 0)),
            pl.BlockSpec((E, D), lambda: (0, 0)),
        ],
        out_specs=(
            pl.BlockSpec((K, T), lambda: (0, 0)),
            pl.BlockSpec((K, T), lambda: (0, 0)),
            pl.BlockSpec((1, 128), lambda: (0, 0)),
            pl.BlockSpec((T, D2), lambda: (0, 0)),
        ),
    )(x, router_w)

    mesh = plsc.VectorSubcoreMesh(core_axis_name="c", subcore_axis_name="s")

    @functools.partial(
        pl.kernel,
        mesh=mesh,
        out_type=jax.ShapeDtypeStruct((S, D2), jnp.int32),
        scratch_types=[
            pltpu.VMEM((TPW,), jnp.int32),
            pltpu.VMEM((TPW,), jnp.int32),
            pltpu.VMEM((TPW, D2), jnp.int32),
            pltpu.SemaphoreType.DMA,
            pltpu.SemaphoreType.DMA,
        ],
    )
    def _dispatch(xb_hbm, slots_hbm, xs_hbm, idx1_v, idx2_v, xbuf, sem1, sem2):
        wid = lax.axis_index("s") * 2 + lax.axis_index("c")
        base = wid * TPW
        pltpu.sync_copy(slots_hbm.at[0, pl.ds(base, TPW)], idx1_v)
        pltpu.sync_copy(slots_hbm.at[1, pl.ds(base, TPW)], idx2_v)
        pltpu.sync_copy(xb_hbm.at[pl.ds(base, TPW)], xbuf)
        c1 = pltpu.async_copy(xbuf, xs_hbm.at[idx1_v], sem1)
        c2 = pltpu.async_copy(xbuf, xs_hbm.at[idx2_v], sem2)
        c1.wait()
        c2.wait()

    xs = _dispatch(xpk, slots)

    grid_spec = pltpu.PrefetchScalarGridSpec(
        num_scalar_prefetch=1,
        grid=(NB,),
        in_specs=[
            pl.BlockSpec(
                (_B, D // 2),
                lambda b, be_s: (jnp.minimum(b, be_s[127] - 1), 0)),
            pl.BlockSpec(
                (1, H2, D),
                lambda b, be_s: (be_s[jnp.minimum(b, be_s[127] - 1)], 0, 0)),
            pl.BlockSpec(
                (1, D, H),
                lambda b, be_s: (be_s[jnp.minimum(b, be_s[127] - 1)], 0, 0)),
            pl.BlockSpec((D,), lambda b, be_s: (0,)),
        ],
        out_specs=pl.BlockSpec(
            (_B, D),
            lambda b, be_s: (jnp.where(b < be_s[127], b, NB - 1), 0)),
    )
    ys = pl.pallas_call(
        functools.partial(_gmm_body, H=H),
        grid_spec=grid_spec,
        out_shape=jax.ShapeDtypeStruct((S, D), jnp.float32),
        compiler_params=pltpu.CompilerParams(
            dimension_semantics=("arbitrary",),
        ),
    )(be.reshape(128), xs, w1, w2, bias)

    @functools.partial(
        pl.kernel,
        mesh=mesh,
        out_type=jax.ShapeDtypeStruct((K, T, D), jnp.float32),
        scratch_types=[
            pltpu.VMEM((TPW,), jnp.int32),
            pltpu.VMEM((TPW,), jnp.int32),
            pltpu.VMEM((TPW, D), jnp.float32),
            pltpu.VMEM((TPW, D), jnp.float32),
            pltpu.SemaphoreType.DMA,
            pltpu.SemaphoreType.DMA,
        ],
    )
    def _unperm(ys_hbm, slots_hbm, yk_hbm, idx1_v, idx2_v, buf1, buf2, sem1, sem2):
        wid = lax.axis_index("s") * 2 + lax.axis_index("c")
        base = wid * TPW
        pltpu.sync_copy(slots_hbm.at[0, pl.ds(base, TPW)], idx1_v)
        pltpu.sync_copy(slots_hbm.at[1, pl.ds(base, TPW)], idx2_v)
        c1 = pltpu.async_copy(ys_hbm.at[idx1_v], buf1, sem1)
        c2 = pltpu.async_copy(ys_hbm.at[idx2_v], buf2, sem2)
        c1.wait()
        pltpu.sync_copy(buf1, yk_hbm.at[0, pl.ds(base, TPW)])
        c2.wait()
        pltpu.sync_copy(buf2, yk_hbm.at[1, pl.ds(base, TPW)])

    yk = _unperm(ys, slots)

    BT = 256
    out = pl.pallas_call(
        _combine_body,
        grid=(T // BT,),
        out_shape=jax.ShapeDtypeStruct((T, D), jnp.float32),
        in_specs=[
            pl.BlockSpec((K, BT, D), lambda t: (0, t, 0)),
            pl.BlockSpec((K, BT), lambda t: (0, t)),
        ],
        out_specs=pl.BlockSpec((BT, D), lambda t: (t, 0)),
    )(yk, wnT)
    return out


# speedup vs baseline: 1.3425x; 1.0780x over previous
import functools

import jax
import jax.numpy as jnp
from jax import lax
from jax.experimental import pallas as pl
from jax.experimental.pallas import tpu as pltpu
from jax.experimental.pallas import tpu_sc as plsc

_B = 576
_NW = 32


def _cumsum_lanes(a, n):
    sh = 1
    while sh < n:
        z = jnp.zeros(a.shape[:1] + (sh,), a.dtype)
        a = a + jnp.concatenate([z, a[:, :-sh]], axis=1)
        sh *= 2
    return a


def _cumsum_subl(a, n):
    sh = 1
    while sh < n:
        z = jnp.zeros((sh,) + a.shape[1:], a.dtype)
        a = a + jnp.concatenate([z, a[:-sh, :]], axis=0)
        sh *= 2
    return a


def _router_body(x_ref, rw_ref, slots_ref, wnT_ref, be_ref, xpk_ref, *, T, E, NB):
    x = x_ref[...]
    Dh = x.shape[1] // 2
    u = lax.bitcast_convert_type(x, jnp.uint32)
    r = (u + jnp.uint32(0x7FFF) + ((u >> 16) & jnp.uint32(1))) >> 16
    xpk_ref[...] = lax.bitcast_convert_type(
        r[:, :Dh] | (r[:, Dh:] << 16), jnp.int32)
    logits = jnp.dot(x, rw_ref[...].T, preferred_element_type=jnp.float32)
    pT = jax.nn.softmax(logits, axis=-1).T
    eiota = lax.broadcasted_iota(jnp.int32, (E, T), 0)
    m1 = jnp.max(pT, axis=0, keepdims=True)
    i1 = jnp.min(jnp.where(pT == m1, eiota, E), axis=0, keepdims=True)
    pm = jnp.where(eiota == i1, -jnp.inf, pT)
    m2 = jnp.max(pm, axis=0, keepdims=True)
    i2 = jnp.min(jnp.where(pm == m2, eiota, E), axis=0, keepdims=True)
    denom = m1 + m2
    wnT_ref[...] = jnp.concatenate([m1 / denom, m2 / denom], axis=0)

    sel = ((eiota == i1) | (eiota == i2)).astype(jnp.int32)
    csum = _cumsum_lanes(sel, T)
    cnt = csum[:, T - 1 : T]
    nblk = (cnt + (_B - 1)) // _B
    blk_incl = _cumsum_subl(nblk, E)
    base = _B * (blk_incl - nblk)
    v = base + csum - sel
    slot1 = jnp.sum(jnp.where(eiota == i1, v, 0), axis=0, keepdims=True)
    slot2 = jnp.sum(jnp.where(eiota == i2, v, 0), axis=0, keepdims=True)
    slots_ref[...] = jnp.concatenate([slot1, slot2], axis=0)

    biota = lax.broadcasted_iota(jnp.int32, (E, 128), 1)
    be = jnp.sum((biota >= blk_incl).astype(jnp.int32), axis=0, keepdims=True)
    be = jnp.minimum(be, E - 1)
    liota = lax.broadcasted_iota(jnp.int32, (1, 128), 1)
    be_ref[...] = jnp.where(liota == 127, blk_incl[E - 1 : E, :], be)


def _gmm_body(be_ref, xs_ref, w1_ref, w2_ref, bias_ref, ys_ref, *, H):
    b = pl.program_id(0)

    @pl.when(b < be_ref[127])
    def _live():
        p = lax.bitcast_convert_type(xs_ref[...], jnp.uint32)
        left = lax.bitcast_convert_type(p << 16, jnp.float32)
        right = lax.bitcast_convert_type(
            p & jnp.uint32(0xFFFF0000), jnp.float32)
        xb = jnp.concatenate([left, right], axis=1)
        h = jnp.dot(xb, w1_ref[0].T, preferred_element_type=jnp.float32)
        g = h[:, :H]
        u = h[:, H:]
        act = g * jax.nn.sigmoid(g) * u
        y = jnp.dot(act, w2_ref[0].T, preferred_element_type=jnp.float32)
        y = y + bias_ref[...][None, :]
        Dh = y.shape[1] // 2
        uy = lax.bitcast_convert_type(y, jnp.uint32)
        r = (uy + jnp.uint32(0x7FFF) + ((uy >> 16) & jnp.uint32(1))) >> 16
        ys_ref[...] = lax.bitcast_convert_type(
            r[:, :Dh] | (r[:, Dh:] << 16), jnp.int32)


def _unpack_bf16_pairs(p_i32):
    p = lax.bitcast_convert_type(p_i32, jnp.uint32)
    left = lax.bitcast_convert_type(p << 16, jnp.float32)
    right = lax.bitcast_convert_type(p & jnp.uint32(0xFFFF0000), jnp.float32)
    return jnp.concatenate([left, right], axis=-1)


def _combine_body(yk_ref, wnT_ref, out_ref):
    a = _unpack_bf16_pairs(yk_ref[0])
    b = _unpack_bf16_pairs(yk_ref[1])
    w1c = wnT_ref[0, :][:, None]
    w2c = wnT_ref[1, :][:, None]
    out_ref[...] = w1c * a + w2c * b


@jax.jit
def kernel(x, w1, w2, router_w, bias):
    T, D = x.shape
    E, H2, _ = w1.shape
    H = H2 // 2
    K = 2
    NB = (T * K + E * (_B - 1)) // _B
    S = NB * _B
    TPW = T // _NW
    HALF = TPW // 2

    D2 = D // 2
    slots, wnT, be, xpk = pl.pallas_call(
        functools.partial(_router_body, T=T, E=E, NB=NB),
        out_shape=(
            jax.ShapeDtypeStruct((K, T), jnp.int32),
            jax.ShapeDtypeStruct((K, T), jnp.float32),
            jax.ShapeDtypeStruct((1, 128), jnp.int32),
            jax.ShapeDtypeStruct((T, D2), jnp.int32),
        ),
        in_specs=[
            pl.BlockSpec((T, D), lambda: (0, 0)),
            pl.BlockSpec((E, D), lambda: (0, 0)),
        ],
        out_specs=(
            pl.BlockSpec((K, T), lambda: (0, 0)),
            pl.BlockSpec((K, T), lambda: (0, 0)),
            pl.BlockSpec((1, 128), lambda: (0, 0)),
            pl.BlockSpec((T, D2), lambda: (0, 0)),
        ),
    )(x, router_w)

    mesh = plsc.VectorSubcoreMesh(core_axis_name="c", subcore_axis_name="s")

    @functools.partial(
        pl.kernel,
        mesh=mesh,
        out_type=jax.ShapeDtypeStruct((S, D2), jnp.int32),
        scratch_types=[
            pltpu.VMEM((TPW,), jnp.int32),
            pltpu.VMEM((TPW,), jnp.int32),
            pltpu.VMEM((TPW, D2), jnp.int32),
            pltpu.SemaphoreType.DMA,
            pltpu.SemaphoreType.DMA,
        ],
    )
    def _dispatch(xb_hbm, slots_hbm, xs_hbm, idx1_v, idx2_v, xbuf, sem1, sem2):
        wid = lax.axis_index("s") * 2 + lax.axis_index("c")
        base = wid * TPW
        pltpu.sync_copy(slots_hbm.at[0, pl.ds(base, TPW)], idx1_v)
        pltpu.sync_copy(slots_hbm.at[1, pl.ds(base, TPW)], idx2_v)
        pltpu.sync_copy(xb_hbm.at[pl.ds(base, TPW)], xbuf)
        c1 = pltpu.async_copy(xbuf, xs_hbm.at[idx1_v], sem1)
        c2 = pltpu.async_copy(xbuf, xs_hbm.at[idx2_v], sem2)
        c1.wait()
        c2.wait()

    xs = _dispatch(xpk, slots)

    grid_spec = pltpu.PrefetchScalarGridSpec(
        num_scalar_prefetch=1,
        grid=(NB,),
        in_specs=[
            pl.BlockSpec(
                (_B, D // 2),
                lambda b, be_s: (jnp.minimum(b, be_s[127] - 1), 0)),
            pl.BlockSpec(
                (1, H2, D),
                lambda b, be_s: (be_s[jnp.minimum(b, be_s[127] - 1)], 0, 0)),
            pl.BlockSpec(
                (1, D, H),
                lambda b, be_s: (be_s[jnp.minimum(b, be_s[127] - 1)], 0, 0)),
            pl.BlockSpec((D,), lambda b, be_s: (0,)),
        ],
        out_specs=pl.BlockSpec(
            (_B, D // 2),
            lambda b, be_s: (jnp.where(b < be_s[127], b, NB - 1), 0)),
    )
    ys = pl.pallas_call(
        functools.partial(_gmm_body, H=H),
        grid_spec=grid_spec,
        out_shape=jax.ShapeDtypeStruct((S, D2), jnp.int32),
        compiler_params=pltpu.CompilerParams(
            dimension_semantics=("arbitrary",),
        ),
    )(be.reshape(128), xs, w1, w2, bias)

    @functools.partial(
        pl.kernel,
        mesh=mesh,
        out_type=jax.ShapeDtypeStruct((K, T, D2), jnp.int32),
        scratch_types=[
            pltpu.VMEM((TPW,), jnp.int32),
            pltpu.VMEM((TPW,), jnp.int32),
            pltpu.VMEM((TPW, D2), jnp.int32),
            pltpu.VMEM((TPW, D2), jnp.int32),
            pltpu.SemaphoreType.DMA,
            pltpu.SemaphoreType.DMA,
        ],
    )
    def _unperm(ys_hbm, slots_hbm, yk_hbm, idx1_v, idx2_v, buf1, buf2, sem1, sem2):
        wid = lax.axis_index("s") * 2 + lax.axis_index("c")
        base = wid * TPW
        pltpu.sync_copy(slots_hbm.at[0, pl.ds(base, TPW)], idx1_v)
        pltpu.sync_copy(slots_hbm.at[1, pl.ds(base, TPW)], idx2_v)
        c1 = pltpu.async_copy(ys_hbm.at[idx1_v], buf1, sem1)
        c2 = pltpu.async_copy(ys_hbm.at[idx2_v], buf2, sem2)
        c1.wait()
        pltpu.sync_copy(buf1, yk_hbm.at[0, pl.ds(base, TPW)])
        c2.wait()
        pltpu.sync_copy(buf2, yk_hbm.at[1, pl.ds(base, TPW)])

    yk = _unperm(ys, slots)

    BT = 256
    out = pl.pallas_call(
        _combine_body,
        grid=(T // BT,),
        out_shape=jax.ShapeDtypeStruct((T, D), jnp.float32),
        in_specs=[
            pl.BlockSpec((K, BT, D2), lambda t: (0, t, 0)),
            pl.BlockSpec((K, BT), lambda t: (0, t)),
        ],
        out_specs=pl.BlockSpec((BT, D), lambda t: (t, 0)),
    )(yk, wnT)
    return out
